# baseline (speedup 1.0000x reference)
import functools

import jax
import jax.numpy as jnp
from jax import lax
from jax.experimental import pallas as pl
from jax.experimental.pallas import tpu as pltpu
from jax.experimental.pallas import tpu_sc as plsc

D = 32
MU = 3.5
L = 16
N_ROWS = 1000000
BLKS = N_ROWS // 128
TAIL0 = BLKS * 128
EPB = 4


@functools.cache
def _build(B: int):
    info = plsc.get_sparse_core_info()
    NC, NS = info.num_cores, info.num_subcores
    NW = NC * NS
    assert B % (8 * NW) == 0
    bpw = B // NW
    groups = bpw // L

    mesh = plsc.VectorSubcoreMesh(core_axis_name="c", subcore_axis_name="s")

    @functools.partial(
        pl.kernel,
        out_type=jax.ShapeDtypeStruct((B,), jnp.float32),
        mesh=mesh,
        compiler_params=pltpu.CompilerParams(
            needs_layout_passes=False, use_tc_tiling_on_sc=True),
        scratch_types=[
            pltpu.VMEM((bpw + L,), jnp.int32),
            pltpu.VMEM((bpw + L,), jnp.int32),
            pltpu.VMEM((2, 2, EPB, D, 128), jnp.float32),
            pltpu.VMEM((16, 128), jnp.float32),
            pltpu.VMEM((16, 128), jnp.float32),
            pltpu.VMEM((bpw,), jnp.float32),
            pltpu.VMEM((bpw,), jnp.float32),
            pltpu.VMEM((bpw,), jnp.float32),
            pltpu.VMEM((32,), jnp.float32),
            pltpu.SemaphoreType.DMA,
            pltpu.SemaphoreType.DMA,
            pltpu.SemaphoreType.DMA,
        ],
    )
    def k(uidx_hbm, iidx_hbm, eut_hbm, eit_hbm, tu_hbm, ti_hbm,
          ub_hbm, ib_hbm, out_hbm,
          uidx_v, iidx_v, stage_v, tu_v, ti_v, ubias_v, ibias_v, out_v,
          tmp_v, ssem, sem0, sem1):
        wid = lax.axis_index("s") * NC + lax.axis_index("c")
        base = wid * bpw
        pltpu.sync_copy(uidx_hbm.at[pl.ds(base, bpw)], uidx_v.at[pl.ds(0, bpw)])
        pltpu.sync_copy(iidx_hbm.at[pl.ds(base, bpw)], iidx_v.at[pl.ds(0, bpw)])
        sc = [
            pltpu.async_copy(ub_hbm.at[uidx_v.at[pl.ds(0, bpw)]], ubias_v, ssem),
            pltpu.async_copy(ib_hbm.at[iidx_v.at[pl.ds(0, bpw)]], ibias_v, ssem),
            pltpu.async_copy(tu_hbm, tu_v, ssem),
            pltpu.async_copy(ti_hbm, ti_v, ssem),
        ]

        iota = lax.iota(jnp.int32, L)
        esel = iota >> 2
        dsub = iota & 3
        zero16 = jnp.zeros((L,), jnp.int32)

        def fire(b, buf, sem):
            iu = uidx_v[pl.ds(b * EPB, L)]
            ii = iidx_v[pl.ds(b * EPB, L)]
            for tbl, iv in ((0, iu), (1, ii)):
                src = eut_hbm if tbl == 0 else eit_hbm
                for e in range(EPB):
                    idx = iv[e]
                    c = jnp.minimum(idx >> 7, BLKS - 1)
                    col = pl.multiple_of(c * 128, 128)
                    pltpu.async_copy(
                        src.at[:, pl.ds(col, 128)],
                        stage_v.at[buf, tbl, e], sem)

        def drain(buf, sem):
            for _ in range(2 * EPB):
                pltpu.make_async_copy(
                    eut_hbm.at[:, pl.ds(0, 128)],
                    stage_v.at[buf, 0, 0], sem).wait()

        def lanevec(iv, fn):
            s = [fn(iv[e]) for e in range(EPB)]
            v = jnp.broadcast_to(s[0], (L,))
            for e in range(1, EPB):
                v = jnp.where(esel == e, s[e], v)
            return v

        def compute(b, buf):
            iu = uidx_v[pl.ds(b * EPB, L)]
            ii = iidx_v[pl.ds(b * EPB, L)]
            lu = lanevec(iu, lambda x: x & 127)
            li = lanevec(ii, lambda x: x & 127)
            tu = lanevec(iu, lambda x: jnp.clip(x - TAIL0, 0, 63))
            ti = lanevec(ii, lambda x: jnp.clip(x - TAIL0, 0, 63))
            mu = lanevec(iu, lambda x: (x >= TAIL0).astype(jnp.int32)) > 0
            mi = lanevec(ii, lambda x: (x >= TAIL0).astype(jnp.int32)) > 0
            bufv = jnp.full((L,), buf, jnp.int32)
            acc = jnp.zeros((L,), jnp.float32)
            for kk in range(8):
                d = dsub + 4 * kk
                uval = plsc.load_gather(
                    stage_v, [bufv, zero16, esel, d, lu])
                ival = plsc.load_gather(
                    stage_v, [bufv, zero16 + 1, esel, d, li])
                fu = tu * D + d
                fi = ti * D + d
                utail = plsc.load_gather(tu_v, [fu >> 7, fu & 127])
                itail = plsc.load_gather(ti_v, [fi >> 7, fi & 127])
                uval = jnp.where(mu, utail, uval)
                ival = jnp.where(mi, itail, ival)
                acc = acc + uval * ival
            p = plsc.cumsum(acc)
            tmp_v[pl.ds(4, L)] = p
            sums = p - tmp_v[pl.ds(0, L)]
            plsc.store_scatter(out_v, [b * EPB + esel], sums, mask=dsub == 3)

        n_batches = bpw // EPB
        tmp_v[pl.ds(0, L)] = jnp.zeros((L,), jnp.float32)
        fire(0, 0, sem0)
        for c in sc:
            c.wait()

        def body(h, carry):
            b0 = 2 * h
            fire(b0 + 1, 1, sem1)
            drain(0, sem0)
            compute(b0, 0)

            @pl.when(b0 + 2 < n_batches)
            def _():
                fire(b0 + 2, 0, sem0)

            drain(1, sem1)
            compute(b0 + 1, 1)
            return carry

        lax.fori_loop(0, n_batches // 2, body, 0)

        def bias_body(g, carry):
            sl = pl.ds(g * L, L)
            out_v[sl] = out_v[sl] + ubias_v[sl] + ibias_v[sl] + MU
            return carry

        lax.fori_loop(0, groups, bias_body, 0)
        pltpu.sync_copy(out_v, out_hbm.at[pl.ds(base, bpw)])

    return k


def kernel(user_idx, item_idx, embed_user, embed_item, user_bias, item_bias):
    B = user_idx.shape[0]
    k = _build(B)
    return k(user_idx.astype(jnp.int32), item_idx.astype(jnp.int32),
             embed_user.T, embed_item.T,
             embed_user[TAIL0:].reshape(16, 128),
             embed_item[TAIL0:].reshape(16, 128),
             jnp.squeeze(user_bias, axis=1), jnp.squeeze(item_bias, axis=1))


# speedup vs baseline: 1.3310x; 1.3310x over previous
import functools

import jax
import jax.numpy as jnp
from jax import lax
from jax.experimental import pallas as pl
from jax.experimental.pallas import tpu as pltpu
from jax.experimental.pallas import tpu_sc as plsc

D = 32
MU = 3.5
L = 16
N_ROWS = 1000000
BLKS = N_ROWS // 128
TAIL0 = BLKS * 128
EPB = 4


@functools.cache
def _build_dots(B: int):
    info = plsc.get_sparse_core_info()
    NC, NS = info.num_cores, info.num_subcores
    NW = NC * NS
    assert B % (8 * NW) == 0
    bpw = B // NW

    mesh = plsc.VectorSubcoreMesh(core_axis_name="c", subcore_axis_name="s")

    @functools.partial(
        pl.kernel,
        out_type=jax.ShapeDtypeStruct((B,), jnp.float32),
        mesh=mesh,
        compiler_params=pltpu.CompilerParams(
            needs_layout_passes=False, use_tc_tiling_on_sc=True),
        scratch_types=[
            pltpu.VMEM((bpw + L,), jnp.int32),
            pltpu.VMEM((bpw + L,), jnp.int32),
            pltpu.VMEM((2, 2, EPB, D, 128), jnp.float32),
            pltpu.VMEM((16, 128), jnp.float32),
            pltpu.VMEM((16, 128), jnp.float32),
            pltpu.VMEM((bpw,), jnp.float32),
            pltpu.VMEM((32,), jnp.float32),
            pltpu.SemaphoreType.DMA,
            pltpu.SemaphoreType.DMA,
            pltpu.SemaphoreType.DMA,
        ],
    )
    def k(uidx_hbm, iidx_hbm, eut_hbm, eit_hbm, tu_hbm, ti_hbm, out_hbm,
          uidx_v, iidx_v, stage_v, tu_v, ti_v, out_v, tmp_v,
          ssem, sem0, sem1):
        wid = lax.axis_index("s") * NC + lax.axis_index("c")
        base = wid * bpw
        pltpu.sync_copy(uidx_hbm.at[pl.ds(base, bpw)], uidx_v.at[pl.ds(0, bpw)])
        pltpu.sync_copy(iidx_hbm.at[pl.ds(base, bpw)], iidx_v.at[pl.ds(0, bpw)])
        sc = [
            pltpu.async_copy(tu_hbm, tu_v, ssem),
            pltpu.async_copy(ti_hbm, ti_v, ssem),
        ]

        iota = lax.iota(jnp.int32, L)
        esel = iota >> 2
        dsub = iota & 3
        zero16 = jnp.zeros((L,), jnp.int32)

        def fire(b, buf, sem):
            iu = uidx_v[pl.ds(b * EPB, L)]
            ii = iidx_v[pl.ds(b * EPB, L)]
            for tbl, iv in ((0, iu), (1, ii)):
                src = eut_hbm if tbl == 0 else eit_hbm
                for e in range(EPB):
                    idx = iv[e]
                    c = jnp.minimum(idx >> 7, BLKS - 1)
                    col = pl.multiple_of(c * 128, 128)
                    pltpu.async_copy(
                        src.at[:, pl.ds(col, 128)],
                        stage_v.at[buf, tbl, e], sem)

        def drain(buf, sem):
            for _ in range(2 * EPB):
                pltpu.make_async_copy(
                    eut_hbm.at[:, pl.ds(0, 128)],
                    stage_v.at[buf, 0, 0], sem).wait()

        def lanevec(iv, fn):
            s = [fn(iv[e]) for e in range(EPB)]
            v = jnp.broadcast_to(s[0], (L,))
            for e in range(1, EPB):
                v = jnp.where(esel == e, s[e], v)
            return v

        def compute(b, buf):
            iu = uidx_v[pl.ds(b * EPB, L)]
            ii = iidx_v[pl.ds(b * EPB, L)]
            lu = lanevec(iu, lambda x: x & 127)
            li = lanevec(ii, lambda x: x & 127)
            tu = lanevec(iu, lambda x: jnp.clip(x - TAIL0, 0, 63))
            ti = lanevec(ii, lambda x: jnp.clip(x - TAIL0, 0, 63))
            mu = lanevec(iu, lambda x: (x >= TAIL0).astype(jnp.int32)) > 0
            mi = lanevec(ii, lambda x: (x >= TAIL0).astype(jnp.int32)) > 0
            bufv = jnp.full((L,), buf, jnp.int32)
            acc = jnp.zeros((L,), jnp.float32)
            for kk in range(8):
                d = dsub + 4 * kk
                uval = plsc.load_gather(
                    stage_v, [bufv, zero16, esel, d, lu])
                ival = plsc.load_gather(
                    stage_v, [bufv, zero16 + 1, esel, d, li])
                fu = tu * D + d
                fi = ti * D + d
                utail = plsc.load_gather(tu_v, [fu >> 7, fu & 127])
                itail = plsc.load_gather(ti_v, [fi >> 7, fi & 127])
                uval = jnp.where(mu, utail, uval)
                ival = jnp.where(mi, itail, ival)
                acc = acc + uval * ival
            p = plsc.cumsum(acc)
            tmp_v[pl.ds(4, L)] = p
            sums = p - tmp_v[pl.ds(0, L)]
            plsc.store_scatter(out_v, [b * EPB + esel], sums, mask=dsub == 3)

        n_batches = bpw // EPB
        tmp_v[pl.ds(0, L)] = jnp.zeros((L,), jnp.float32)
        fire(0, 0, sem0)
        for c in sc:
            c.wait()

        def body(h, carry):
            b0 = 2 * h
            fire(b0 + 1, 1, sem1)
            drain(0, sem0)
            compute(b0, 0)

            @pl.when(b0 + 2 < n_batches)
            def _():
                fire(b0 + 2, 0, sem0)

            drain(1, sem1)
            compute(b0 + 1, 1)
            return carry

        lax.fori_loop(0, n_batches // 2, body, 0)
        pltpu.sync_copy(out_v, out_hbm.at[pl.ds(base, bpw)])

    return k


@functools.cache
def _build_bias(B: int):
    info = plsc.get_sparse_core_info()
    NC, NS = info.num_cores, info.num_subcores
    NW = NC * NS
    bpw = B // NW
    groups = bpw // L

    mesh = plsc.VectorSubcoreMesh(core_axis_name="c", subcore_axis_name="s")

    @functools.partial(
        pl.kernel,
        out_type=jax.ShapeDtypeStruct((B,), jnp.float32),
        mesh=mesh,
        compiler_params=pltpu.CompilerParams(
            needs_layout_passes=False, use_tc_tiling_on_sc=True),
        scratch_types=[
            pltpu.VMEM((bpw,), jnp.int32),
            pltpu.VMEM((bpw,), jnp.int32),
            pltpu.VMEM((bpw,), jnp.float32),
            pltpu.VMEM((bpw,), jnp.float32),
            pltpu.VMEM((bpw,), jnp.float32),
            pltpu.VMEM((bpw,), jnp.float32),
            pltpu.SemaphoreType.DMA,
        ],
    )
    def k(dots_hbm, uidx_hbm, iidx_hbm, ub_hbm, ib_hbm, out_hbm,
          uidx_v, iidx_v, dots_v, ubias_v, ibias_v, out_v, sem):
        wid = lax.axis_index("s") * NC + lax.axis_index("c")
        base = wid * bpw
        pltpu.sync_copy(uidx_hbm.at[pl.ds(base, bpw)], uidx_v)
        pltpu.sync_copy(iidx_hbm.at[pl.ds(base, bpw)], iidx_v)
        cs = [
            pltpu.async_copy(ub_hbm.at[uidx_v], ubias_v, sem),
            pltpu.async_copy(ib_hbm.at[iidx_v], ibias_v, sem),
            pltpu.async_copy(dots_hbm.at[pl.ds(base, bpw)], dots_v, sem),
        ]
        for c in cs:
            c.wait()

        def body(g, carry):
            sl = pl.ds(g * L, L)
            out_v[sl] = dots_v[sl] + ubias_v[sl] + ibias_v[sl] + MU
            return carry

        lax.fori_loop(0, groups, body, 0)
        pltpu.sync_copy(out_v, out_hbm.at[pl.ds(base, bpw)])

    return k


def kernel(user_idx, item_idx, embed_user, embed_item, user_bias, item_bias):
    B = user_idx.shape[0]
    uidx = user_idx.astype(jnp.int32)
    iidx = item_idx.astype(jnp.int32)
    dots = _build_dots(B)(
        uidx, iidx, embed_user.T, embed_item.T,
        embed_user[TAIL0:].reshape(16, 128),
        embed_item[TAIL0:].reshape(16, 128))
    return _build_bias(B)(
        dots, uidx, iidx,
        jnp.squeeze(user_bias, axis=1), jnp.squeeze(item_bias, axis=1))
